# baseline (speedup 1.0000x reference)
import functools

import jax
import jax.numpy as jnp
from jax import lax
from jax.experimental import pallas as pl
from jax.experimental.pallas import tpu as pltpu
from jax.experimental.pallas import tpu_sc as plsc

N = 10000
D = 128
E = 320000
N_PAD = 10240
E_PAD = 327680
ROWS_PER_TILE = N_PAD // 16
ROW_CHUNKS = ROWS_PER_TILE // 128
IDX_ROWS = E_PAD // 128
CHUNKS_PER_TILE = IDX_ROWS // 16

def _rsqrt16(d):
    g = jnp.full((16,), 2.0 ** (-0.25), jnp.float32)
    for j in range(1, 21):
        g = jnp.where(d >= 2.0 ** j, 2.0 ** (-j / 2 - 0.25), g)
    for _ in range(3):
        g = g * (1.5 - 0.5 * d * g * g)
    return jnp.where(d > 0.5, g, 0.0)


SLAB = 16
N_SLABS = CHUNKS_PER_TILE // SLAB


def _splat(vec16, lane):
    idx = jnp.full((16, 1), 0, dtype=jnp.int32) + lane
    dn = lax.GatherDimensionNumbers(
        offset_dims=(), collapsed_slice_dims=(0,), start_index_map=(0,))
    return lax.gather(vec16, idx, dn, slice_sizes=(1,),
                      mode=lax.GatherScatterMode.PROMISE_IN_BOUNDS)


def _scale_rows(buf, dv_v, dv_base, nrows):
    @pl.loop(0, nrows)
    def _row(i):
        g16 = (i >> 4) << 4
        dv16 = dv_v[pl.ds(dv_base + g16, 16)]
        s16 = _splat(dv16, i - g16)
        for v in range(8):
            sl = pl.ds(v * 16, 16)
            buf[i, sl] = buf[i, sl] * s16


def _scale_rows_128(buf, dv_v, dv_base):
    _scale_rows(buf, dv_v, dv_base, 128)


def _scale_stream(src, dv_v, out_hbm, r0, rbuf0, rbuf1, sg0, sg1):
    bufs, sems = (rbuf0, rbuf1), (sg0, sg1)

    def sl(k):
        return (pl.ds(r0 + k * 128, 128), slice(None))

    pltpu.async_copy(src.at[sl(0)], rbuf0, sg0)
    for k in range(ROW_CHUNKS):
        b, s = bufs[k % 2], sems[k % 2]
        pltpu.make_async_copy(src.at[sl(k)], b, s).wait()
        _scale_rows_128(b, dv_v, k * 128)
        pltpu.async_copy(b, out_hbm.at[sl(k)], s)
        if k + 1 < ROW_CHUNKS:
            b2, s2 = bufs[(k + 1) % 2], sems[(k + 1) % 2]
            if k >= 1:
                pltpu.make_async_copy(b2, out_hbm.at[sl(k - 1)], s2).wait()
            pltpu.async_copy(src.at[sl(k + 1)], b2, s2)
    for k in (ROW_CHUNKS - 2, ROW_CHUNKS - 1):
        pltpu.make_async_copy(bufs[k % 2], out_hbm.at[sl(k)],
                              sems[k % 2]).wait()


def _run_direction(src_hbm, sidx_hbm, didx_hbm, stage_hbm, out_hbm, t,
                   acc, deg_s, deg_d, idx_s, idx_d, idx_s2, idx_d2,
                   rbuf0, rbuf1, dvs_v, dvd_v, ones_v, sg0, sg1, si0, si1, ss0, ss1, sh):
    zeros16 = jnp.zeros((16,), jnp.float32)
    r0 = t * ROWS_PER_TILE
    c0 = t * CHUNKS_PER_TILE

    @pl.loop(0, 128)
    def _zero_rbuf(i):
        for v in range(8):
            rbuf0[i, pl.ds(v * 16, 16)] = zeros16

    for k in range(ROW_CHUNKS):
        pltpu.async_copy(rbuf0, acc.at[pl.ds(r0 + k * 128, 128), :], sg0)

    @pl.loop(0, ROWS_PER_TILE // 16)
    def _zero_deg(i):
        dvs_v[pl.ds(i * 16, 16)] = zeros16

    pltpu.async_copy(dvs_v, deg_s.at[pl.ds(r0, ROWS_PER_TILE)], sg1)
    pltpu.async_copy(dvs_v, deg_d.at[pl.ds(r0, ROWS_PER_TILE)], sg1)

    for v in range(8):
        ones_v[pl.ds(v * 16, 16)] = zeros16 + 1.0

    for k in range(ROW_CHUNKS):
        pltpu.make_async_copy(rbuf0, acc.at[pl.ds(r0 + k * 128, 128), :],
                              sg0).wait()
    pltpu.make_async_copy(dvs_v, deg_s.at[pl.ds(r0, ROWS_PER_TILE)],
                          sg1).wait()
    pltpu.make_async_copy(dvs_v, deg_d.at[pl.ds(r0, ROWS_PER_TILE)],
                          sg1).wait()

    plsc.subcore_barrier()

    @pl.loop(0, N_SLABS)
    def _hist_slab(sb):
        row = c0 + sb * SLAB
        pltpu.sync_copy(sidx_hbm.at[pl.ds(row, SLAB), :], idx_s)
        pltpu.sync_copy(didx_hbm.at[pl.ds(row, SLAB), :], idx_d)

        @pl.loop(0, SLAB)
        def _hist_fire(j):
            pltpu.async_copy(ones_v, deg_s.at[idx_s.at[j]], sg0, add=True)
            pltpu.async_copy(ones_v, deg_d.at[idx_d.at[j]], sg1, add=True)

        @pl.loop(0, SLAB)
        def _hist_drain(j):
            pltpu.make_async_copy(ones_v, deg_s.at[idx_s.at[j]], sg0).wait()
            pltpu.make_async_copy(ones_v, deg_d.at[idx_d.at[j]], sg1).wait()

    plsc.subcore_barrier()

    pltpu.sync_copy(deg_s.at[pl.ds(r0, ROWS_PER_TILE)], dvs_v)
    pltpu.sync_copy(deg_d.at[pl.ds(r0, ROWS_PER_TILE)], dvd_v)

    @pl.loop(0, ROWS_PER_TILE // 16)
    def _dinv(i):
        sl = pl.ds(i * 16, 16)
        dvs_v[sl] = _rsqrt16(dvs_v[sl])
        dvd_v[sl] = _rsqrt16(dvd_v[sl])

    @pl.when(t < 15)
    def _p3_full():
        _scale_stream(src_hbm, dvs_v, stage_hbm, r0, rbuf0, rbuf1, sg0, sg1)

    @pl.when(t == 15)
    def _p3_tail():
        for k in range(3):
            s3 = (pl.ds(r0 + k * 128, 128), slice(None))
            pltpu.sync_copy(src_hbm.at[s3], rbuf0)
            _scale_rows_128(rbuf0, dvs_v, k * 128)
            pltpu.sync_copy(rbuf0, stage_hbm.at[s3])
        sp = (pl.ds(r0 + 384, 16), slice(None))
        pltpu.sync_copy(src_hbm.at[sp], rbuf0.at[pl.ds(0, 16), :])
        _scale_rows_128(rbuf0, dvs_v, 384)
        pltpu.sync_copy(rbuf0.at[pl.ds(0, 16), :], stage_hbm.at[sp])

    plsc.subcore_barrier()

    def load_idx(sb, i_s, i_d, sem):
        row = c0 + sb * SLAB
        pltpu.async_copy(sidx_hbm.at[pl.ds(row, SLAB), :], i_s, sem)
        pltpu.async_copy(didx_hbm.at[pl.ds(row, SLAB), :], i_d, sem)

    def wait_idx(sb, i_s, i_d, sem):
        row = c0 + sb * SLAB
        pltpu.make_async_copy(sidx_hbm.at[pl.ds(row, SLAB), :], i_s,
                              sem).wait()
        pltpu.make_async_copy(didx_hbm.at[pl.ds(row, SLAB), :], i_d,
                              sem).wait()

    def start_g(i_s, j, buf, sem):
        pltpu.async_copy(stage_hbm.at[i_s.at[j]], buf, sem)

    def wait_g(i_s, j, buf, sem):
        pltpu.make_async_copy(stage_hbm.at[i_s.at[j]], buf, sem).wait()

    def scat(i_d, j, buf):
        pltpu.sync_copy(buf, acc.at[i_d.at[j]], add=True)

    def process_slab(i_s, i_d, boundary0, boundary1):
        @pl.loop(0, SLAB // 2 - 1)
        def _main(p):
            j0 = 2 * p
            wait_g(i_s, j0, rbuf0, sg0)
            scat(i_d, j0, rbuf0)
            start_g(i_s, j0 + 2, rbuf0, sg0)
            wait_g(i_s, j0 + 1, rbuf1, sg1)
            scat(i_d, j0 + 1, rbuf1)
            start_g(i_s, j0 + 3, rbuf1, sg1)

        wait_g(i_s, SLAB - 2, rbuf0, sg0)
        scat(i_d, SLAB - 2, rbuf0)
        boundary0()
        wait_g(i_s, SLAB - 1, rbuf1, sg1)
        scat(i_d, SLAB - 1, rbuf1)
        boundary1()

    load_idx(0, idx_s, idx_d, si0)
    load_idx(1, idx_s2, idx_d2, si1)
    wait_idx(0, idx_s, idx_d, si0)
    start_g(idx_s, 0, rbuf0, sg0)
    start_g(idx_s, 1, rbuf1, sg1)

    @pl.loop(0, N_SLABS // 2)
    def _main_pair(q):
        sb0 = 2 * q
        last = N_SLABS // 2 - 1

        def _eb0():
            wait_idx(sb0 + 1, idx_s2, idx_d2, si1)
            start_g(idx_s2, 0, rbuf0, sg0)

        def _eb1():
            start_g(idx_s2, 1, rbuf1, sg1)

        process_slab(idx_s, idx_d, _eb0, _eb1)

        @pl.when(q < last)
        def _prefetch_even():
            load_idx(sb0 + 2, idx_s, idx_d, si0)

        def _ob0():
            @pl.when(q < last)
            def _():
                wait_idx(sb0 + 2, idx_s, idx_d, si0)
                start_g(idx_s, 0, rbuf0, sg0)

        def _ob1():
            @pl.when(q < last)
            def _():
                start_g(idx_s, 1, rbuf1, sg1)

        process_slab(idx_s2, idx_d2, _ob0, _ob1)

        @pl.when(q < last)
        def _prefetch_odd():
            load_idx(sb0 + 3, idx_s2, idx_d2, si1)

    plsc.subcore_barrier()

    @pl.when(t < 15)
    def _p5_full():
        _scale_stream(acc, dvd_v, out_hbm, r0, rbuf0, rbuf1, sg0, sg1)

    @pl.when(t == 15)
    def _p5_tail():
        for k in range(3):
            s5 = (pl.ds(r0 + k * 128, 128), slice(None))
            pltpu.sync_copy(acc.at[s5], rbuf0)
            _scale_rows_128(rbuf0, dvd_v, k * 128)
            pltpu.sync_copy(rbuf0, out_hbm.at[s5])
        sp = (pl.ds(r0 + 384, 16), slice(None))
        pltpu.sync_copy(acc.at[sp], rbuf0.at[pl.ds(0, 16), :])
        _scale_rows_128(rbuf0, dvd_v, 384)
        pltpu.sync_copy(rbuf0.at[pl.ds(0, 16), :], out_hbm.at[sp])


def _sc_body(x_ref, y_ref, fr_ref, to_ref, o_yx, o_xy, stg0, stg1,
             acc, deg_s, deg_d, idx_s, idx_d, idx_s2, idx_d2, rbuf0, rbuf1,
             dvs_v, dvd_v, ones_v, sg0, sg1, si0, si1, ss0, ss1, sh):
    c = lax.axis_index("c")
    t = lax.axis_index("s")
    scratch = (acc, deg_s, deg_d, idx_s, idx_d, idx_s2, idx_d2, rbuf0, rbuf1,
               dvs_v, dvd_v, ones_v, sg0, sg1, si0, si1, ss0, ss1, sh)

    @pl.when(c == 0)
    def _():
        _run_direction(x_ref, fr_ref, to_ref, stg0, o_xy, t, *scratch)

    @pl.when(c == 1)
    def _():
        _run_direction(y_ref, to_ref, fr_ref, stg1, o_yx, t, *scratch)


@functools.partial(jax.jit, static_argnums=())
def _propagate(x_pad, y_pad, fr2d, to2d):
    f32 = jnp.float32
    kfn = pl.kernel(
        _sc_body,
        out_type=(
            jax.ShapeDtypeStruct((N, D), f32),
            jax.ShapeDtypeStruct((N, D), f32),
            jax.ShapeDtypeStruct((N_PAD, D), f32),
            jax.ShapeDtypeStruct((N_PAD, D), f32),
        ),
        mesh=plsc.VectorSubcoreMesh(core_axis_name="c", subcore_axis_name="s"),
        scratch_types=(
            pltpu.VMEM_SHARED((N_PAD, D), f32),
            pltpu.VMEM_SHARED((N_PAD,), f32),
            pltpu.VMEM_SHARED((N_PAD,), f32),
            pltpu.VMEM((SLAB, 128), jnp.int32),
            pltpu.VMEM((SLAB, 128), jnp.int32),
            pltpu.VMEM((SLAB, 128), jnp.int32),
            pltpu.VMEM((SLAB, 128), jnp.int32),
            pltpu.VMEM((128, D), f32),
            pltpu.VMEM((128, D), f32),
            pltpu.VMEM((ROWS_PER_TILE,), f32),
            pltpu.VMEM((ROWS_PER_TILE,), f32),
            pltpu.VMEM((128,), f32),
            pltpu.SemaphoreType.DMA,
            pltpu.SemaphoreType.DMA,
            pltpu.SemaphoreType.DMA,
            pltpu.SemaphoreType.DMA,
            pltpu.SemaphoreType.DMA,
            pltpu.SemaphoreType.DMA,
            pltpu.SemaphoreType.DMA,
        ),
    )
    return kfn(x_pad, y_pad, fr2d, to2d)[:2]


def kernel(x, y, edge_index):
    pad_e = E_PAD - edge_index.shape[1]
    pad_idx = N + (jnp.arange(pad_e, dtype=jnp.int32) % (N_PAD - N))
    fr = jnp.concatenate([edge_index[0], pad_idx]).reshape(IDX_ROWS, 128)
    to = jnp.concatenate([edge_index[1], pad_idx]).reshape(IDX_ROWS, 128)
    y2x, x2y = _propagate(x, y, fr, to)
    return (y2x, x2y)


# speedup vs baseline: 1.0162x; 1.0162x over previous
import functools

import jax
import jax.numpy as jnp
from jax import lax
from jax.experimental import pallas as pl
from jax.experimental.pallas import tpu as pltpu
from jax.experimental.pallas import tpu_sc as plsc

N = 10000
D = 128
E = 320000
N_PAD = 10240
E_PAD = 327680
ROWS_PER_TILE = N_PAD // 16
ROW_CHUNKS = ROWS_PER_TILE // 128
IDX_ROWS = E_PAD // 128
CHUNKS_PER_TILE = IDX_ROWS // 16

def _rsqrt16(d):
    g = jnp.full((16,), 2.0 ** (-0.25), jnp.float32)
    for j in range(1, 21):
        g = jnp.where(d >= 2.0 ** j, 2.0 ** (-j / 2 - 0.25), g)
    for _ in range(3):
        g = g * (1.5 - 0.5 * d * g * g)
    return jnp.where(d > 0.5, g, 0.0)


SLAB = 16
N_SLABS = CHUNKS_PER_TILE // SLAB


def _splat(vec16, lane):
    idx = jnp.full((16, 1), 0, dtype=jnp.int32) + lane
    dn = lax.GatherDimensionNumbers(
        offset_dims=(), collapsed_slice_dims=(0,), start_index_map=(0,))
    return lax.gather(vec16, idx, dn, slice_sizes=(1,),
                      mode=lax.GatherScatterMode.PROMISE_IN_BOUNDS)


def _scale_rows(buf, dv_v, dv_base, nrows):
    @pl.loop(0, nrows)
    def _row(i):
        g16 = (i >> 4) << 4
        dv16 = dv_v[pl.ds(dv_base + g16, 16)]
        s16 = _splat(dv16, i - g16)
        for v in range(8):
            sl = pl.ds(v * 16, 16)
            buf[i, sl] = buf[i, sl] * s16


def _scale_rows_128(buf, dv_v, dv_base):
    @pl.loop(0, 8)
    def _grp(g):
        dv16 = dv_v[pl.ds(dv_base + g * 16, 16)]
        for i in range(16):
            s16 = _splat(dv16, i)
            for v in range(8):
                sl = pl.ds(v * 16, 16)
                buf[g * 16 + i, sl] = buf[g * 16 + i, sl] * s16


def _scale_stream(src, dv_v, out_hbm, r0, rbuf0, rbuf1, sg0, sg1):
    bufs, sems = (rbuf0, rbuf1), (sg0, sg1)

    def sl(k):
        return (pl.ds(r0 + k * 128, 128), slice(None))

    pltpu.async_copy(src.at[sl(0)], rbuf0, sg0)
    for k in range(ROW_CHUNKS):
        b, s = bufs[k % 2], sems[k % 2]
        pltpu.make_async_copy(src.at[sl(k)], b, s).wait()
        _scale_rows_128(b, dv_v, k * 128)
        pltpu.async_copy(b, out_hbm.at[sl(k)], s)
        if k + 1 < ROW_CHUNKS:
            b2, s2 = bufs[(k + 1) % 2], sems[(k + 1) % 2]
            if k >= 1:
                pltpu.make_async_copy(b2, out_hbm.at[sl(k - 1)], s2).wait()
            pltpu.async_copy(src.at[sl(k + 1)], b2, s2)
    for k in (ROW_CHUNKS - 2, ROW_CHUNKS - 1):
        pltpu.make_async_copy(bufs[k % 2], out_hbm.at[sl(k)],
                              sems[k % 2]).wait()


def _run_direction(src_hbm, sidx_hbm, didx_hbm, stage_hbm, out_hbm, t,
                   acc, deg_s, deg_d, idx_s, idx_d, idx_s2, idx_d2,
                   rbuf0, rbuf1, dvs_v, dvd_v, ones_v, sg0, sg1, si0, si1, ss0, ss1, sh):
    zeros16 = jnp.zeros((16,), jnp.float32)
    r0 = t * ROWS_PER_TILE
    c0 = t * CHUNKS_PER_TILE

    @pl.loop(0, 128)
    def _zero_rbuf(i):
        for v in range(8):
            rbuf0[i, pl.ds(v * 16, 16)] = zeros16

    for k in range(ROW_CHUNKS):
        pltpu.async_copy(rbuf0, acc.at[pl.ds(r0 + k * 128, 128), :], sg0)

    @pl.loop(0, ROWS_PER_TILE // 16)
    def _zero_deg(i):
        dvs_v[pl.ds(i * 16, 16)] = zeros16

    pltpu.async_copy(dvs_v, deg_s.at[pl.ds(r0, ROWS_PER_TILE)], sg1)
    pltpu.async_copy(dvs_v, deg_d.at[pl.ds(r0, ROWS_PER_TILE)], sg1)

    for v in range(8):
        ones_v[pl.ds(v * 16, 16)] = zeros16 + 1.0

    for k in range(ROW_CHUNKS):
        pltpu.make_async_copy(rbuf0, acc.at[pl.ds(r0 + k * 128, 128), :],
                              sg0).wait()
    pltpu.make_async_copy(dvs_v, deg_s.at[pl.ds(r0, ROWS_PER_TILE)],
                          sg1).wait()
    pltpu.make_async_copy(dvs_v, deg_d.at[pl.ds(r0, ROWS_PER_TILE)],
                          sg1).wait()

    plsc.subcore_barrier()

    @pl.loop(0, N_SLABS)
    def _hist_slab(sb):
        row = c0 + sb * SLAB
        pltpu.sync_copy(sidx_hbm.at[pl.ds(row, SLAB), :], idx_s)
        pltpu.sync_copy(didx_hbm.at[pl.ds(row, SLAB), :], idx_d)

        @pl.loop(0, SLAB)
        def _hist_fire(j):
            pltpu.async_copy(ones_v, deg_s.at[idx_s.at[j]], sg0, add=True)
            pltpu.async_copy(ones_v, deg_d.at[idx_d.at[j]], sg1, add=True)

        @pl.loop(0, SLAB)
        def _hist_drain(j):
            pltpu.make_async_copy(ones_v, deg_s.at[idx_s.at[j]], sg0).wait()
            pltpu.make_async_copy(ones_v, deg_d.at[idx_d.at[j]], sg1).wait()

    plsc.subcore_barrier()

    pltpu.sync_copy(deg_s.at[pl.ds(r0, ROWS_PER_TILE)], dvs_v)
    pltpu.sync_copy(deg_d.at[pl.ds(r0, ROWS_PER_TILE)], dvd_v)

    @pl.loop(0, ROWS_PER_TILE // 16)
    def _dinv(i):
        sl = pl.ds(i * 16, 16)
        dvs_v[sl] = _rsqrt16(dvs_v[sl])
        dvd_v[sl] = _rsqrt16(dvd_v[sl])

    @pl.when(t < 15)
    def _p3_full():
        _scale_stream(src_hbm, dvs_v, stage_hbm, r0, rbuf0, rbuf1, sg0, sg1)

    @pl.when(t == 15)
    def _p3_tail():
        for k in range(3):
            s3 = (pl.ds(r0 + k * 128, 128), slice(None))
            pltpu.sync_copy(src_hbm.at[s3], rbuf0)
            _scale_rows_128(rbuf0, dvs_v, k * 128)
            pltpu.sync_copy(rbuf0, stage_hbm.at[s3])
        sp = (pl.ds(r0 + 384, 16), slice(None))
        pltpu.sync_copy(src_hbm.at[sp], rbuf0.at[pl.ds(0, 16), :])
        _scale_rows_128(rbuf0, dvs_v, 384)
        pltpu.sync_copy(rbuf0.at[pl.ds(0, 16), :], stage_hbm.at[sp])

    plsc.subcore_barrier()

    def load_idx(sb, i_s, i_d, sem):
        row = c0 + sb * SLAB
        pltpu.async_copy(sidx_hbm.at[pl.ds(row, SLAB), :], i_s, sem)
        pltpu.async_copy(didx_hbm.at[pl.ds(row, SLAB), :], i_d, sem)

    def wait_idx(sb, i_s, i_d, sem):
        row = c0 + sb * SLAB
        pltpu.make_async_copy(sidx_hbm.at[pl.ds(row, SLAB), :], i_s,
                              sem).wait()
        pltpu.make_async_copy(didx_hbm.at[pl.ds(row, SLAB), :], i_d,
                              sem).wait()

    def start_g(i_s, j, buf, sem):
        pltpu.async_copy(stage_hbm.at[i_s.at[j]], buf, sem)

    def wait_g(i_s, j, buf, sem):
        pltpu.make_async_copy(stage_hbm.at[i_s.at[j]], buf, sem).wait()

    def scat(i_d, j, buf):
        pltpu.sync_copy(buf, acc.at[i_d.at[j]], add=True)

    def process_slab(i_s, i_d, boundary0, boundary1):
        @pl.loop(0, SLAB // 2 - 1)
        def _main(p):
            j0 = 2 * p
            wait_g(i_s, j0, rbuf0, sg0)
            scat(i_d, j0, rbuf0)
            start_g(i_s, j0 + 2, rbuf0, sg0)
            wait_g(i_s, j0 + 1, rbuf1, sg1)
            scat(i_d, j0 + 1, rbuf1)
            start_g(i_s, j0 + 3, rbuf1, sg1)

        wait_g(i_s, SLAB - 2, rbuf0, sg0)
        scat(i_d, SLAB - 2, rbuf0)
        boundary0()
        wait_g(i_s, SLAB - 1, rbuf1, sg1)
        scat(i_d, SLAB - 1, rbuf1)
        boundary1()

    load_idx(0, idx_s, idx_d, si0)
    load_idx(1, idx_s2, idx_d2, si1)
    wait_idx(0, idx_s, idx_d, si0)
    start_g(idx_s, 0, rbuf0, sg0)
    start_g(idx_s, 1, rbuf1, sg1)

    @pl.loop(0, N_SLABS // 2)
    def _main_pair(q):
        sb0 = 2 * q
        last = N_SLABS // 2 - 1

        def _eb0():
            wait_idx(sb0 + 1, idx_s2, idx_d2, si1)
            start_g(idx_s2, 0, rbuf0, sg0)

        def _eb1():
            start_g(idx_s2, 1, rbuf1, sg1)

        process_slab(idx_s, idx_d, _eb0, _eb1)

        @pl.when(q < last)
        def _prefetch_even():
            load_idx(sb0 + 2, idx_s, idx_d, si0)

        def _ob0():
            @pl.when(q < last)
            def _():
                wait_idx(sb0 + 2, idx_s, idx_d, si0)
                start_g(idx_s, 0, rbuf0, sg0)

        def _ob1():
            @pl.when(q < last)
            def _():
                start_g(idx_s, 1, rbuf1, sg1)

        process_slab(idx_s2, idx_d2, _ob0, _ob1)

        @pl.when(q < last)
        def _prefetch_odd():
            load_idx(sb0 + 3, idx_s2, idx_d2, si1)

    plsc.subcore_barrier()

    @pl.when(t < 15)
    def _p5_full():
        _scale_stream(acc, dvd_v, out_hbm, r0, rbuf0, rbuf1, sg0, sg1)

    @pl.when(t == 15)
    def _p5_tail():
        for k in range(3):
            s5 = (pl.ds(r0 + k * 128, 128), slice(None))
            pltpu.sync_copy(acc.at[s5], rbuf0)
            _scale_rows_128(rbuf0, dvd_v, k * 128)
            pltpu.sync_copy(rbuf0, out_hbm.at[s5])
        sp = (pl.ds(r0 + 384, 16), slice(None))
        pltpu.sync_copy(acc.at[sp], rbuf0.at[pl.ds(0, 16), :])
        _scale_rows_128(rbuf0, dvd_v, 384)
        pltpu.sync_copy(rbuf0.at[pl.ds(0, 16), :], out_hbm.at[sp])


def _sc_body(x_ref, y_ref, fr_ref, to_ref, o_yx, o_xy, stg0, stg1,
             acc, deg_s, deg_d, idx_s, idx_d, idx_s2, idx_d2, rbuf0, rbuf1,
             dvs_v, dvd_v, ones_v, sg0, sg1, si0, si1, ss0, ss1, sh):
    c = lax.axis_index("c")
    t = lax.axis_index("s")
    scratch = (acc, deg_s, deg_d, idx_s, idx_d, idx_s2, idx_d2, rbuf0, rbuf1,
               dvs_v, dvd_v, ones_v, sg0, sg1, si0, si1, ss0, ss1, sh)

    @pl.when(c == 0)
    def _():
        _run_direction(x_ref, fr_ref, to_ref, stg0, o_xy, t, *scratch)

    @pl.when(c == 1)
    def _():
        _run_direction(y_ref, to_ref, fr_ref, stg1, o_yx, t, *scratch)


@functools.partial(jax.jit, static_argnums=())
def _propagate(x_pad, y_pad, fr2d, to2d):
    f32 = jnp.float32
    kfn = pl.kernel(
        _sc_body,
        out_type=(
            jax.ShapeDtypeStruct((N, D), f32),
            jax.ShapeDtypeStruct((N, D), f32),
            jax.ShapeDtypeStruct((N_PAD, D), f32),
            jax.ShapeDtypeStruct((N_PAD, D), f32),
        ),
        mesh=plsc.VectorSubcoreMesh(core_axis_name="c", subcore_axis_name="s"),
        scratch_types=(
            pltpu.VMEM_SHARED((N_PAD, D), f32),
            pltpu.VMEM_SHARED((N_PAD,), f32),
            pltpu.VMEM_SHARED((N_PAD,), f32),
            pltpu.VMEM((SLAB, 128), jnp.int32),
            pltpu.VMEM((SLAB, 128), jnp.int32),
            pltpu.VMEM((SLAB, 128), jnp.int32),
            pltpu.VMEM((SLAB, 128), jnp.int32),
            pltpu.VMEM((128, D), f32),
            pltpu.VMEM((128, D), f32),
            pltpu.VMEM((ROWS_PER_TILE,), f32),
            pltpu.VMEM((ROWS_PER_TILE,), f32),
            pltpu.VMEM((128,), f32),
            pltpu.SemaphoreType.DMA,
            pltpu.SemaphoreType.DMA,
            pltpu.SemaphoreType.DMA,
            pltpu.SemaphoreType.DMA,
            pltpu.SemaphoreType.DMA,
            pltpu.SemaphoreType.DMA,
            pltpu.SemaphoreType.DMA,
        ),
    )
    return kfn(x_pad, y_pad, fr2d, to2d)[:2]


def kernel(x, y, edge_index):
    pad_e = E_PAD - edge_index.shape[1]
    pad_idx = N + (jnp.arange(pad_e, dtype=jnp.int32) % (N_PAD - N))
    fr = jnp.concatenate([edge_index[0], pad_idx]).reshape(IDX_ROWS, 128)
    to = jnp.concatenate([edge_index[1], pad_idx]).reshape(IDX_ROWS, 128)
    y2x, x2y = _propagate(x, y, fr, to)
    return (y2x, x2y)
